# baseline (speedup 1.0000x reference)
import functools

import jax
import jax.numpy as jnp
from jax.experimental import pallas as pl
from jax.experimental.pallas import tpu as pltpu


def _center_loss_kernel(feat_ref, lt1_ref, lt2_ref, c1_ref, c2_ref, out_ref,
                        acc1_ref, acc2_ref, *, inv_scale, nsteps, ncols, nvalid1, nvalid2):
    i = pl.program_id(0)
    f = feat_ref[...]
    fb = f.astype(jnp.bfloat16)
    f2 = jnp.sum(f * f, axis=1, keepdims=True)
    g = jnp.concatenate(
        [fb, f2.astype(jnp.bfloat16), jnp.ones_like(fb[:, :1])], axis=1)

    def half(lt_ref, acc_ref):
        lab = lt_ref[...].astype(jnp.bfloat16)
        m = jax.lax.dot_general(lab, g, (((1,), (0,)), ((), ())),
                                preferred_element_type=jnp.float32
                                ).astype(jnp.bfloat16)

        @pl.when(i == 0)
        def _():
            acc_ref[...] = m

        @pl.when(i > 0)
        def _():
            acc_ref[...] += m

    half(lt1_ref, acc1_ref)
    half(lt2_ref, acc2_ref)

    @pl.when(i == nsteps - 1)
    def _():
        def tail(c_ref, acc_ref, nvalid):
            c = c_ref[...]
            acc = acc_ref[...].astype(jnp.float32)
            ch = acc.shape[0]
            if nvalid < ch:
                rows = jax.lax.broadcasted_iota(jnp.int32, (ch, 1), 0)
                keep = rows < nvalid
                acc = jnp.where(keep, acc, 0.0)
                c = jnp.where(keep, c, 0.0)
            c2 = jnp.sum(c * c, axis=1)
            t12 = jnp.sum(acc[:, ncols - 2]) + jnp.sum(c2 * acc[:, ncols - 1])
            t3 = jnp.sum(c * acc[:, :ncols - 2])
            return t12 - 2.0 * t3

        out_ref[0, 0] = (tail(c1_ref, acc1_ref, nvalid1)
                         + tail(c2_ref, acc2_ref, nvalid2)) * inv_scale


def kernel(feat, label, centers):
    B, D = feat.shape
    C = label.shape[1]
    lt = label.T
    TB = 1024 if B % 1024 == 0 else B
    nsteps = B // TB
    ncols = D + 2
    CH = 512
    out = pl.pallas_call(
        functools.partial(_center_loss_kernel,
                          inv_scale=1.0 / (2.0 * B * C),
                          nsteps=nsteps, ncols=ncols,
                          nvalid1=min(CH, C), nvalid2=C - CH),
        grid=(nsteps,),
        in_specs=[
            pl.BlockSpec((TB, D), lambda i: (i, 0)),
            pl.BlockSpec((CH, TB), lambda i: (0, i)),
            pl.BlockSpec((CH, TB), lambda i: (1, i)),
            pl.BlockSpec((CH, D), lambda i: (0, 0)),
            pl.BlockSpec((CH, D), lambda i: (1, 0)),
        ],
        out_specs=pl.BlockSpec((1, 1), lambda i: (0, 0), memory_space=pltpu.SMEM),
        out_shape=jax.ShapeDtypeStruct((1, 1), jnp.float32),
        scratch_shapes=[
            pltpu.VMEM((CH, ncols), jnp.bfloat16),
            pltpu.VMEM((CH, ncols), jnp.bfloat16),
        ],
    )(feat, lt, lt, centers, centers)
    return out[0, 0]
